# baseline (speedup 1.0000x reference)
import jax
import jax.numpy as jnp
from jax import lax
from jax.experimental import pallas as pl
from jax.experimental.pallas import tpu as pltpu
from jax.experimental.pallas import tpu_sc as plsc

B, H, S, D = 8, 16, 4096, 128
Q = 16
BH = B * H

NC, NS = 2, 16
NW = NC * NS
BH_PER_W = BH // NW
ROWS_PER_W = BH_PER_W * S

NBUF = 4
CH = 128
NCHUNK = ROWS_PER_W // CH


def _tc_k_body(pos_ref, kval_ref, kc_ref, ko_ref):
    ko_ref[...] = kc_ref[...]
    for q in range(Q):
        p = pos_ref[q]
        m = q
        for r in range(q + 1, Q):
            m = jnp.where(pos_ref[r] == p, r, m)
        ko_ref[0, pl.ds(p, 1), :] = kval_ref[0, pl.ds(m, 1), :]


def _tc_k(pos, kv, kc):
    cache_spec = pl.BlockSpec((1, S, D), lambda i: (i, 0, 0))
    val_spec = pl.BlockSpec((1, Q, D), lambda i: (i, 0, 0))
    return pl.pallas_call(
        _tc_k_body,
        grid=(BH,),
        in_specs=[pl.BlockSpec(memory_space=pltpu.SMEM), val_spec, cache_spec],
        out_specs=cache_spec,
        out_shape=jax.ShapeDtypeStruct((BH, S, D), jnp.float32),
        compiler_params=pltpu.CompilerParams(
            dimension_semantics=("arbitrary",),
        ),
    )(pos, kv, kc)


def _sc_v_body(pos_hbm, vval_hbm, vc_hbm, vo_hbm,
               pos_v, src_v, dst_v, vrows,
               b0, b1, b2, b3, si0, si1, si2, si3, so0, so1, so2, so3, sem):
    wid = lax.axis_index("s") * NC + lax.axis_index("c")
    row0 = wid * ROWS_PER_W
    bufs = (b0, b1, b2, b3)
    sin = (si0, si1, si2, si3)
    sout = (so0, so1, so2, so3)

    for b in range(NBUF):
        pltpu.async_copy(vc_hbm.at[pl.ds(row0 + b * CH, CH)], bufs[b], sin[b])

    @pl.loop(0, NCHUNK, step=NBUF)
    def _(g):
        for b in range(NBUF):
            c = g + b
            pltpu.make_async_copy(vc_hbm.at[pl.ds(row0 + c * CH, CH)],
                                  bufs[b], sin[b]).wait()
            pltpu.async_copy(bufs[b], vo_hbm.at[pl.ds(row0 + c * CH, CH)],
                             sout[b])
        for b in range(NBUF):
            c = g + b
            pltpu.make_async_copy(bufs[b],
                                  vo_hbm.at[pl.ds(row0 + c * CH, CH)],
                                  sout[b]).wait()

            @pl.when(c + NBUF < NCHUNK)
            def _():
                pltpu.async_copy(vc_hbm.at[pl.ds(row0 + (c + NBUF) * CH, CH)],
                                 bufs[b], sin[b])

    pltpu.sync_copy(pos_hbm, pos_v)
    pos = pos_v[...]
    iota = lax.iota(jnp.int32, 16)
    m = iota
    for r in range(1, Q):
        pos_r = jnp.take_along_axis(pos, jnp.full((Q,), r, jnp.int32), axis=0)
        m = jnp.where(pos == pos_r, r, m)

    for j in range(BH_PER_W):
        bh = wid * BH_PER_W + j
        src_v[pl.ds(j * Q, Q)] = bh * Q + m
        dst_v[pl.ds(j * Q, Q)] = bh * S + pos

    pltpu.async_copy(vval_hbm.at[src_v], vrows, sem).wait()
    pltpu.async_copy(vrows, vo_hbm.at[dst_v], sem).wait()


_sc_v = pl.kernel(
    _sc_v_body,
    out_type=jax.ShapeDtypeStruct((BH * S, D), jnp.float32),
    mesh=plsc.VectorSubcoreMesh(core_axis_name="c", subcore_axis_name="s"),
    scratch_types=(
        [
            pltpu.VMEM((Q,), jnp.int32),
            pltpu.VMEM((BH_PER_W * Q,), jnp.int32),
            pltpu.VMEM((BH_PER_W * Q,), jnp.int32),
            pltpu.VMEM((BH_PER_W * Q, D), jnp.float32),
        ]
        + [pltpu.VMEM((CH, D), jnp.float32) for _ in range(NBUF)]
        + [pltpu.SemaphoreType.DMA for _ in range(2 * NBUF + 1)]
    ),
)


def kernel(input_pos, k_val, v_val, k_cache, v_cache):
    ko = _tc_k(input_pos, k_val.reshape(BH, Q, D), k_cache.reshape(BH, S, D))
    vo = _sc_v(input_pos, v_val.reshape(BH * Q, D), v_cache.reshape(BH * S, D))
    return ko.reshape(B, H, S, D), vo.reshape(B, H, S, D)
